# baseline (speedup 1.0000x reference)
import functools
import math

import jax
import jax.numpy as jnp
from jax import lax
from jax.experimental import pallas as pl
from jax.experimental.pallas import tpu as pltpu
from jax.experimental.pallas import tpu_sc as plsc

B = 1024
D = 64
NUM_FIXED = 128
MAX_VARS = 512
SCALE = 1.0 / math.sqrt(D)

NC = 2
NS = 16
NW = NC * NS
BPW = B // NW
L = 16
CH = 128
NBUF = 4
TW = 2 * D
TOTCH = BPW * MAX_VARS // CH
OWORDS = BPW * MAX_VARS


def _dense_body(q_ref, kv_ref, kp_ref, fx_ref, vr_ref):
    qb = q_ref[...]
    fx_ref[...] = lax.dot_general(
        qb, kp_ref[...], (((1,), (1,)), ((), ())),
        preferred_element_type=jnp.float32,
        precision=lax.Precision.HIGHEST) * SCALE
    kv = kv_ref[...]
    vr_ref[...] = jnp.sum(kv * qb[:, None, :], axis=-1) * SCALE


def _dense_parts(q, k_var, k_param):
    BB = 64
    grid = (B // BB,)
    return pl.pallas_call(
        _dense_body,
        grid=grid,
        in_specs=[
            pl.BlockSpec((BB, D), lambda i: (i, 0)),
            pl.BlockSpec((BB, MAX_VARS, D), lambda i: (i, 0, 0)),
            pl.BlockSpec((NUM_FIXED, D), lambda i: (0, 0)),
        ],
        out_specs=[
            pl.BlockSpec((BB, NUM_FIXED), lambda i: (i, 0)),
            pl.BlockSpec((BB, MAX_VARS), lambda i: (i, 0)),
        ],
        out_shape=[
            jax.ShapeDtypeStruct((B, NUM_FIXED), jnp.float32),
            jax.ShapeDtypeStruct((B, MAX_VARS), jnp.float32),
        ],
    )(q, k_var, k_param)


def _sc_body(table_hbm, args_hbm, q_hbm, out_hbm,
             argbuf0, argbuf1, idx_c, jg_c, q_all,
             ring0, ring1, ring2, ring3, out_all,
             sem_arg, sem_q, sem_ring):
    wid = lax.axis_index("s") * NC + lax.axis_index("c")
    base = wid * BPW
    iota = lax.iota(jnp.int32, L)
    argbufs = (argbuf0, argbuf1)
    rings = (ring0, ring1, ring2, ring3)

    def arg_copy(i, slot):
        return pltpu.make_async_copy(
            args_hbm.at[base + i], argbufs[slot], sem_arg.at[slot])

    def chunk_copy(k, slot):
        return pltpu.make_async_copy(
            table_hbm.at[idx_c.at[pl.ds(k * CH, CH)]],
            rings[slot], sem_ring.at[slot])

    qcp = pltpu.make_async_copy(
        q_hbm.at[pl.ds(base * D, BPW * D)], q_all, sem_q)
    qcp.start()
    arg_copy(0, 0).start()

    def zero_phase(z, carry):
        for t in range(16):
            out_all[pl.ds(z * 256 + t * L, L)] = jnp.zeros((L,), jnp.float32)
        return carry

    lax.fori_loop(0, (OWORDS + 256) // 256, zero_phase, 0, unroll=False)

    def idx_phase(io, cnt):
        for i2 in range(2):
            i = io * 2 + i2
            arg_copy(i, i2).wait()
            if i2 == 0:
                arg_copy(i + 1, 1).start()
            else:
                @pl.when(io < BPW // 2 - 1)
                def _():
                    arg_copy(i + 1, 0).start()
            ab = argbufs[i2]
            for t in range(MAX_VARS // L):
                ev = iota * 2 + (2 * t * L)
                a0 = plsc.load_gather(ab, [ev])
                a1 = plsc.load_gather(ab, [ev + 1])
                ok = a0 >= 0
                idx = a0 * MAX_VARS + a1
                jg = (i * MAX_VARS + t * L + iota) + ((idx & 1) << 15)
                plsc.store_compressed(idx_c.at[pl.ds(cnt, L)], idx >> 1, mask=ok)
                plsc.store_compressed(jg_c.at[pl.ds(cnt, L)], jg, mask=ok)
                cnt = cnt + plsc.all_reduce_population_count(ok)[0]
        return cnt

    cnt = lax.fori_loop(0, BPW // 2, idx_phase, 0, unroll=False)
    for w in range(CH // L):
        idx_c[pl.ds(cnt + w * L, L)] = jnp.zeros((L,), jnp.int32)
        jg_c[pl.ds(cnt + w * L, L)] = OWORDS + iota
    nchunks = (cnt + CH - 1) // CH
    qcp.wait()

    for kk in range(NBUF):
        @pl.when(kk < nchunks)
        def _(_kk=kk):
            chunk_copy(_kk, _kk).start()

    def main_phase(ko, carry):
        for kslot in range(NBUF):
            k = ko * NBUF + kslot

            @pl.when(k < nchunks)
            def _(_kslot=kslot, _k=k):
                chunk_copy(_k, _kslot).wait()

                def body_g(g, carry2):
                    rowbase = g * L + iota
                    comb = jg_c[pl.ds(_k * CH + g * L, L)]
                    jg = comb & (2 ** 15 - 1)
                    lobase = (comb >> 15) << 6
                    qbase = ((jg >> 9) << 6) & (OWORDS // 8 - 1)
                    acc = jnp.zeros((L,), jnp.float32)
                    for s in range(D):
                        col = (iota + s) & (D - 1)
                        vals = plsc.load_gather(
                            rings[_kslot], [rowbase, lobase + col])
                        qv = plsc.load_gather(q_all, [qbase + col])
                        acc = acc + vals * qv
                    plsc.store_scatter(out_all, [jg], acc * SCALE)
                    return carry2

                lax.fori_loop(0, CH // L, body_g, 0, unroll=False)

                @pl.when(_k + NBUF < nchunks)
                def _():
                    chunk_copy(_k + NBUF, _kslot).start()
        return carry

    lax.fori_loop(0, TOTCH // NBUF, main_phase, 0, unroll=False)

    pltpu.sync_copy(out_all.at[pl.ds(0, OWORDS)],
                    out_hbm.at[pl.ds(base * MAX_VARS, OWORDS)])


def _arg_scores(k_arg_param, args_flat, q_flat):
    mesh = plsc.VectorSubcoreMesh(core_axis_name="c", subcore_axis_name="s")
    kern = pl.kernel(
        _sc_body,
        out_type=jax.ShapeDtypeStruct((B * MAX_VARS,), jnp.float32),
        mesh=mesh,
        compiler_params=pltpu.CompilerParams(
            needs_layout_passes=False, use_tc_tiling_on_sc=True),
        scratch_types=[
            pltpu.VMEM((2 * MAX_VARS,), jnp.int32),
            pltpu.VMEM((2 * MAX_VARS,), jnp.int32),
            pltpu.VMEM((OWORDS + CH,), jnp.int32),
            pltpu.VMEM((OWORDS + CH,), jnp.int32),
            pltpu.VMEM((BPW * D,), jnp.float32),
            pltpu.VMEM((CH, TW), jnp.float32),
            pltpu.VMEM((CH, TW), jnp.float32),
            pltpu.VMEM((CH, TW), jnp.float32),
            pltpu.VMEM((CH, TW), jnp.float32),
            pltpu.VMEM((OWORDS + 256,), jnp.float32),
            pltpu.SemaphoreType.DMA((2,)),
            pltpu.SemaphoreType.DMA,
            pltpu.SemaphoreType.DMA((NBUF,)),
        ],
    )
    return kern(k_arg_param, args_flat, q_flat)


def kernel(q, k_var, args, k_param, k_arg_param):
    args_flat = args.reshape(B, 2 * MAX_VARS)
    fx, vr = _dense_parts(q, k_var, k_param)
    ar = _arg_scores(k_arg_param.reshape(-1, 2 * D), args_flat, q.reshape(-1))
    ar = ar.reshape(B, MAX_VARS)
    return jnp.concatenate([fx, vr, ar], axis=1)


# speedup vs baseline: 1.1170x; 1.1170x over previous
import functools
import math

import jax
import jax.numpy as jnp
from jax import lax
from jax.experimental import pallas as pl
from jax.experimental.pallas import tpu as pltpu
from jax.experimental.pallas import tpu_sc as plsc

B = 1024
D = 64
NUM_FIXED = 128
MAX_VARS = 512
SCALE = 1.0 / math.sqrt(D)

NC = 2
NS = 16
NW = NC * NS
BPW = B // NW
L = 16
CH = 128
NBUF = 8
TOTCH = BPW * MAX_VARS // CH
OWORDS = BPW * MAX_VARS


def _dense_body(q_ref, kv_ref, kp_ref, fx_ref, vr_ref):
    qb = q_ref[...]
    fx_ref[...] = lax.dot_general(
        qb, kp_ref[...], (((1,), (1,)), ((), ())),
        preferred_element_type=jnp.float32,
        precision=lax.Precision.HIGHEST) * SCALE
    kv = kv_ref[...]
    vr_ref[...] = jnp.sum(kv * qb[:, None, :], axis=-1) * SCALE


def _dense_parts(q, k_var, k_param):
    BB = 64
    grid = (B // BB,)
    return pl.pallas_call(
        _dense_body,
        grid=grid,
        in_specs=[
            pl.BlockSpec((BB, D), lambda i: (i, 0)),
            pl.BlockSpec((BB, MAX_VARS, D), lambda i: (i, 0, 0)),
            pl.BlockSpec((NUM_FIXED, D), lambda i: (0, 0)),
        ],
        out_specs=[
            pl.BlockSpec((BB, NUM_FIXED), lambda i: (i, 0)),
            pl.BlockSpec((BB, MAX_VARS), lambda i: (i, 0)),
        ],
        out_shape=[
            jax.ShapeDtypeStruct((B, NUM_FIXED), jnp.float32),
            jax.ShapeDtypeStruct((B, MAX_VARS), jnp.float32),
        ],
    )(q, k_var, k_param)


def _sc_body(table_hbm, args_hbm, q_hbm, out_hbm,
             argbuf, idx_c, jg_c, q_all, ring, out_all,
             sem_arg, sem_q, sem_ring):
    wid = lax.axis_index("s") * NC + lax.axis_index("c")
    base = wid * BPW
    iota = lax.iota(jnp.int32, L)

    def arg_copy(i, slot):
        return pltpu.make_async_copy(
            args_hbm.at[base + i], argbuf.at[slot], sem_arg.at[slot])

    def chunk_copy(k, slot):
        return pltpu.make_async_copy(
            table_hbm.at[idx_c.at[pl.ds(k * CH, CH)]],
            ring.at[slot], sem_ring.at[slot])

    qcp = pltpu.make_async_copy(
        q_hbm.at[pl.ds(base * D, BPW * D)], q_all, sem_q)
    qcp.start()
    arg_copy(0, 0).start()

    def zero_phase(z, carry):
        for t in range(16):
            out_all[pl.ds(z * 256 + t * L, L)] = jnp.zeros((L,), jnp.float32)
        return carry

    lax.fori_loop(0, (OWORDS + 256) // 256, zero_phase, 0, unroll=False)

    def idx_phase(io, cnt):
        for i2 in range(2):
            i = io * 2 + i2
            arg_copy(i, i2).wait()
            if i2 == 0:
                arg_copy(i + 1, 1).start()
            else:
                @pl.when(io < BPW // 2 - 1)
                def _():
                    arg_copy(i + 1, 0).start()
            ab = argbuf.at[i2]
            for t in range(MAX_VARS // L):
                ev = iota * 2 + (2 * t * L)
                a0 = plsc.load_gather(ab, [ev])
                a1 = plsc.load_gather(ab, [ev + 1])
                ok = a0 >= 0
                idx = a0 * MAX_VARS + a1
                jg = i * MAX_VARS + t * L + iota
                plsc.store_compressed(idx_c.at[pl.ds(cnt, L)], idx, mask=ok)
                plsc.store_compressed(jg_c.at[pl.ds(cnt, L)], jg, mask=ok)
                cnt = cnt + plsc.all_reduce_population_count(ok)[0]
        return cnt

    cnt = lax.fori_loop(0, BPW // 2, idx_phase, 0, unroll=False)
    for w in range(CH // L):
        idx_c[pl.ds(cnt + w * L, L)] = jnp.zeros((L,), jnp.int32)
        jg_c[pl.ds(cnt + w * L, L)] = OWORDS + iota
    nchunks = (cnt + CH - 1) // CH
    qcp.wait()

    for kk in range(NBUF):
        @pl.when(kk < nchunks)
        def _(_kk=kk):
            chunk_copy(_kk, _kk).start()

    def main_phase(ko, carry):
        for kslot in range(NBUF):
            k = ko * NBUF + kslot

            @pl.when(k < nchunks)
            def _(_kslot=kslot, _k=k):
                chunk_copy(_k, _kslot).wait()

                def body_g(g, carry2):
                    rowbase = g * L + iota
                    jg = jg_c[pl.ds(_k * CH + g * L, L)]
                    qbase = ((jg >> 9) << 6) & (OWORDS // 8 - 1)
                    acc = jnp.zeros((L,), jnp.float32)
                    for s in range(D):
                        col = (iota + s) & (D - 1)
                        vals = plsc.load_gather(
                            ring.at[_kslot], [rowbase, col])
                        qv = plsc.load_gather(q_all, [qbase + col])
                        acc = acc + vals * qv
                    plsc.store_scatter(out_all, [jg], acc * SCALE)
                    return carry2

                lax.fori_loop(0, CH // L, body_g, 0, unroll=False)

                @pl.when(_k + NBUF < nchunks)
                def _():
                    chunk_copy(_k + NBUF, _kslot).start()
        return carry

    lax.fori_loop(0, TOTCH // NBUF, main_phase, 0, unroll=False)

    pltpu.sync_copy(out_all.at[pl.ds(0, OWORDS)],
                    out_hbm.at[pl.ds(base * MAX_VARS, OWORDS)])


def _arg_scores(k_arg_param, args_flat, q_flat):
    mesh = plsc.VectorSubcoreMesh(core_axis_name="c", subcore_axis_name="s")
    kern = pl.kernel(
        _sc_body,
        out_type=jax.ShapeDtypeStruct((B * MAX_VARS,), jnp.float32),
        mesh=mesh,
        compiler_params=pltpu.CompilerParams(
            needs_layout_passes=False, use_tc_tiling_on_sc=False),
        scratch_types=[
            pltpu.VMEM((2, 2 * MAX_VARS), jnp.int32),
            pltpu.VMEM((OWORDS + CH,), jnp.int32),
            pltpu.VMEM((OWORDS + CH,), jnp.int32),
            pltpu.VMEM((BPW * D,), jnp.float32),
            pltpu.VMEM((NBUF, CH, D), jnp.float32),
            pltpu.VMEM((OWORDS + 256,), jnp.float32),
            pltpu.SemaphoreType.DMA((2,)),
            pltpu.SemaphoreType.DMA,
            pltpu.SemaphoreType.DMA((NBUF,)),
        ],
    )
    return kern(k_arg_param, args_flat, q_flat)


def kernel(q, k_var, args, k_param, k_arg_param):
    args_flat = args.reshape(B, 2 * MAX_VARS)
    fx, vr = _dense_parts(q, k_var, k_param)
    ar = _arg_scores(k_arg_param, args_flat, q.reshape(-1))
    ar = ar.reshape(B, MAX_VARS)
    return jnp.concatenate([fx, vr, ar], axis=1)


# speedup vs baseline: 1.1818x; 1.0580x over previous
import functools
import math

import jax
import jax.numpy as jnp
from jax import lax
from jax.experimental import pallas as pl
from jax.experimental.pallas import tpu as pltpu
from jax.experimental.pallas import tpu_sc as plsc

B = 1024
D = 64
NUM_FIXED = 128
MAX_VARS = 512
SCALE = 1.0 / math.sqrt(D)

NC = 2
NS = 16
NW = NC * NS
BPW = B // NW
L = 16
CH = 64
NBUF = 16
TOTCH = BPW * MAX_VARS // CH
OWORDS = BPW * MAX_VARS


def _dense_body(q_ref, kv_ref, kp_ref, fx_ref, vr_ref):
    qb = q_ref[...]
    fx_ref[...] = lax.dot_general(
        qb, kp_ref[...], (((1,), (1,)), ((), ())),
        preferred_element_type=jnp.float32,
        precision=lax.Precision.HIGHEST) * SCALE
    kv = kv_ref[...]
    vr_ref[...] = jnp.sum(kv * qb[:, None, :], axis=-1) * SCALE


def _dense_parts(q, k_var, k_param):
    BB = 64
    grid = (B // BB,)
    return pl.pallas_call(
        _dense_body,
        grid=grid,
        in_specs=[
            pl.BlockSpec((BB, D), lambda i: (i, 0)),
            pl.BlockSpec((BB, MAX_VARS, D), lambda i: (i, 0, 0)),
            pl.BlockSpec((NUM_FIXED, D), lambda i: (0, 0)),
        ],
        out_specs=[
            pl.BlockSpec((BB, NUM_FIXED), lambda i: (i, 0)),
            pl.BlockSpec((BB, MAX_VARS), lambda i: (i, 0)),
        ],
        out_shape=[
            jax.ShapeDtypeStruct((B, NUM_FIXED), jnp.float32),
            jax.ShapeDtypeStruct((B, MAX_VARS), jnp.float32),
        ],
    )(q, k_var, k_param)


def _sc_body(table_hbm, args_hbm, q_hbm, out_hbm,
             argbuf, idx_c, jg_c, q_all, ring, out_all,
             sem_arg, sem_q, sem_ring):
    wid = lax.axis_index("s") * NC + lax.axis_index("c")
    base = wid * BPW
    iota = lax.iota(jnp.int32, L)

    def arg_copy(i, slot):
        return pltpu.make_async_copy(
            args_hbm.at[base + i], argbuf.at[slot], sem_arg.at[slot])

    def chunk_copy(k, slot):
        return pltpu.make_async_copy(
            table_hbm.at[idx_c.at[pl.ds(k * CH, CH)]],
            ring.at[slot], sem_ring.at[slot])

    qcp = pltpu.make_async_copy(
        q_hbm.at[pl.ds(base * D, BPW * D)], q_all, sem_q)
    qcp.start()
    arg_copy(0, 0).start()

    def zero_phase(z, carry):
        for t in range(16):
            out_all[pl.ds(z * 256 + t * L, L)] = jnp.zeros((L,), jnp.float32)
        return carry

    lax.fori_loop(0, (OWORDS + 256) // 256, zero_phase, 0, unroll=False)

    def idx_phase(io, cnt):
        for i2 in range(2):
            i = io * 2 + i2
            arg_copy(i, i2).wait()
            if i2 == 0:
                arg_copy(i + 1, 1).start()
            else:
                @pl.when(io < BPW // 2 - 1)
                def _():
                    arg_copy(i + 1, 0).start()
            ab = argbuf.at[i2]
            for t in range(MAX_VARS // L):
                ev = iota * 2 + (2 * t * L)
                a0 = plsc.load_gather(ab, [ev])
                a1 = plsc.load_gather(ab, [ev + 1])
                ok = a0 >= 0
                idx = a0 * MAX_VARS + a1
                jg = i * MAX_VARS + t * L + iota
                plsc.store_compressed(idx_c.at[pl.ds(cnt, L)], idx, mask=ok)
                plsc.store_compressed(jg_c.at[pl.ds(cnt, L)], jg, mask=ok)
                cnt = cnt + plsc.all_reduce_population_count(ok)[0]
        return cnt

    cnt = lax.fori_loop(0, BPW // 2, idx_phase, 0, unroll=False)
    for w in range(CH // L):
        idx_c[pl.ds(cnt + w * L, L)] = jnp.zeros((L,), jnp.int32)
        jg_c[pl.ds(cnt + w * L, L)] = OWORDS + iota
    nchunks = (cnt + CH - 1) // CH
    qcp.wait()

    for kk in range(NBUF):
        @pl.when(kk < nchunks)
        def _(_kk=kk):
            chunk_copy(_kk, _kk).start()

    def main_phase(ko, carry):
        for kslot in range(NBUF):
            k = ko * NBUF + kslot

            @pl.when(k < nchunks)
            def _(_kslot=kslot, _k=k):
                chunk_copy(_k, _kslot).wait()

                def body_g(g, carry2):
                    rowbase = g * L + iota
                    jg = jg_c[pl.ds(_k * CH + g * L, L)]
                    qbase = ((jg >> 9) << 6) & (OWORDS // 8 - 1)
                    acc = jnp.zeros((L,), jnp.float32)
                    for s in range(D):
                        col = (iota + s) & (D - 1)
                        vals = plsc.load_gather(
                            ring.at[_kslot], [rowbase, col])
                        qv = plsc.load_gather(q_all, [qbase + col])
                        acc = acc + vals * qv
                    plsc.store_scatter(out_all, [jg], acc * SCALE)
                    return carry2

                lax.fori_loop(0, CH // L, body_g, 0, unroll=False)

                @pl.when(_k + NBUF < nchunks)
                def _():
                    chunk_copy(_k + NBUF, _kslot).start()
        return carry

    lax.fori_loop(0, TOTCH // NBUF, main_phase, 0, unroll=False)

    pltpu.sync_copy(out_all.at[pl.ds(0, OWORDS)],
                    out_hbm.at[pl.ds(base * MAX_VARS, OWORDS)])


def _arg_scores(k_arg_param, args_flat, q_flat):
    mesh = plsc.VectorSubcoreMesh(core_axis_name="c", subcore_axis_name="s")
    kern = pl.kernel(
        _sc_body,
        out_type=jax.ShapeDtypeStruct((B * MAX_VARS,), jnp.float32),
        mesh=mesh,
        compiler_params=pltpu.CompilerParams(
            needs_layout_passes=False, use_tc_tiling_on_sc=False),
        scratch_types=[
            pltpu.VMEM((2, 2 * MAX_VARS), jnp.int32),
            pltpu.VMEM((OWORDS + CH,), jnp.int32),
            pltpu.VMEM((OWORDS + CH,), jnp.int32),
            pltpu.VMEM((BPW * D,), jnp.float32),
            pltpu.VMEM((NBUF, CH, D), jnp.float32),
            pltpu.VMEM((OWORDS + 256,), jnp.float32),
            pltpu.SemaphoreType.DMA((2,)),
            pltpu.SemaphoreType.DMA,
            pltpu.SemaphoreType.DMA((NBUF,)),
        ],
    )
    return kern(k_arg_param, args_flat, q_flat)


def kernel(q, k_var, args, k_param, k_arg_param):
    args_flat = args.reshape(B, 2 * MAX_VARS)
    fx, vr = _dense_parts(q, k_var, k_param)
    ar = _arg_scores(k_arg_param, args_flat, q.reshape(-1))
    ar = ar.reshape(B, MAX_VARS)
    return jnp.concatenate([fx, vr, ar], axis=1)
